# baseline (speedup 1.0000x reference)
import functools

import jax
import jax.numpy as jnp
from jax import lax
from jax.experimental import pallas as pl
from jax.experimental.pallas import tpu as pltpu
from jax.experimental.pallas import tpu_sc as plsc

_N = 10000
_E = 320000
_D = 128
_G = 64
_NB = 3
_HID = 64
_NCLS = 10

_BN = 1000
_NBLK = _N // _BN

_NC = 2
_NS = 16
_NW = _NC * _NS
_K = 80
_EPW = _E // _NW
_NCH = _EPW // _K
_NCH2 = (_NCH - 1) // 2
_BE = 2000
_EBLK = _E // _BE
_NP = 10112
_RPT = _NP // _NS
_RCH = 128
_F32 = jnp.float32


def _mm(a, b):
    return jnp.dot(a, b, preferred_element_type=_F32,
                   precision=jax.lax.Precision.HIGHEST)


def _lrelu(v):
    return jnp.where(v > 0, v, 0.01 * v)


def _fc_body(x_ref, w_ref, b_ref, o_ref):
    o_ref[...] = _mm(x_ref[...], w_ref[...]) + b_ref[...]


_fc_call = pl.pallas_call(
    _fc_body,
    grid=(_NBLK,),
    in_specs=[
        pl.BlockSpec((_BN, _D), lambda i: (i, 0)),
        pl.BlockSpec((_D, _D), lambda i: (0, 0)),
        pl.BlockSpec((1, _D), lambda i: (0, 0)),
    ],
    out_specs=pl.BlockSpec((_BN, _D), lambda i: (i, 0)),
    out_shape=jax.ShapeDtypeStruct((_N, _D), _F32),
)


def _ro_body(p0, p1, b3, gf, rw1, rw2, rb, w1a, w1b, a_o, b_o, gfn_o, st_o):
    i = pl.program_id(0)
    xb = p0[...] + p1[...]
    bt = b3[0, 0, :]
    oh = (bt[:, None] == lax.broadcasted_iota(jnp.int32, (_BN, _G), 1)).astype(_F32)
    oht = (lax.broadcasted_iota(jnp.int32, (_G, _BN), 0) == bt[None, :]).astype(_F32)
    gfb = _mm(oh, gf[...])
    gw = jax.nn.sigmoid(_mm(xb, rw1[...]) + _mm(gfb, rw2[...]) + rb[...])
    gfn_c = _mm(oht, gw * xb)
    nns = jnp.sqrt(jnp.sum(gw * gw, axis=1, keepdims=True))
    li = lax.broadcasted_iota(jnp.int32, (_BN, 8), 1)
    s8 = jnp.where(li == 0, nns, jnp.where(li == 1, 1.0, 0.0))
    st_c = _mm(oht, s8)
    a_o[...] = _mm(xb, w1a[...])
    b_o[...] = _mm(xb, w1b[...])

    @pl.when(i == 0)
    def _():
        gfn_o[...] = gfn_c
        st_o[...] = st_c

    @pl.when(i != 0)
    def _():
        gfn_o[...] += gfn_c
        st_o[...] += st_c


_ro_call = pl.pallas_call(
    _ro_body,
    grid=(_NBLK,),
    in_specs=[
        pl.BlockSpec((_BN, _D), lambda i: (i, 0)),
        pl.BlockSpec((_BN, _D), lambda i: (i, 0)),
        pl.BlockSpec((1, 1, _BN), lambda i: (i, 0, 0)),
        pl.BlockSpec((_G, _D), lambda i: (0, 0)),
        pl.BlockSpec((_D, _D), lambda i: (0, 0)),
        pl.BlockSpec((_D, _D), lambda i: (0, 0)),
        pl.BlockSpec((1, _D), lambda i: (0, 0)),
        pl.BlockSpec((_D, _D), lambda i: (0, 0)),
        pl.BlockSpec((_D, _D), lambda i: (0, 0)),
    ],
    out_specs=[
        pl.BlockSpec((_BN, _D), lambda i: (i, 0)),
        pl.BlockSpec((_BN, _D), lambda i: (i, 0)),
        pl.BlockSpec((_G, _D), lambda i: (0, 0)),
        pl.BlockSpec((_G, 8), lambda i: (0, 0)),
    ],
    out_shape=[
        jax.ShapeDtypeStruct((_N, _D), _F32),
        jax.ShapeDtypeStruct((_N, _D), _F32),
        jax.ShapeDtypeStruct((_G, _D), _F32),
        jax.ShapeDtypeStruct((_G, 8), _F32),
    ],
)


def _gru_body(gfn, gf, st, wih, bih, whh, bhh, w1c, b1, gf2_o, c2_o, ggl_o):
    gi = _mm(gfn[...], wih[...]) + bih[...]
    gh = _mm(gf[...], whh[...]) + bhh[...]
    r = jax.nn.sigmoid(gi[:, :_D] + gh[:, :_D])
    z = jax.nn.sigmoid(gi[:, _D:2 * _D] + gh[:, _D:2 * _D])
    n = jnp.tanh(gi[:, 2 * _D:] + r * gh[:, 2 * _D:])
    g2 = (1.0 - z) * n + z * gf[...]
    gf2_o[...] = g2
    c2_o[...] = _mm(g2, w1c[...]) + b1[...]
    s = st[...]
    val = jnp.mean(s[:, 0:1] / jnp.maximum(s[:, 1:2], 1.0))
    ggl_o[...] = val * jnp.ones((1, 1), _F32)


_gru_call = pl.pallas_call(
    _gru_body,
    out_shape=[
        jax.ShapeDtypeStruct((_G, _D), _F32),
        jax.ShapeDtypeStruct((_G, _D), _F32),
        jax.ShapeDtypeStruct((1, 1), _F32),
    ],
)


def _acn_body(a, b3, c2, o):
    bt = b3[0, 0, :]
    oh = (bt[:, None] == lax.broadcasted_iota(jnp.int32, (_BN, _G), 1)).astype(_F32)
    o[...] = a[...] + _mm(oh, c2[...])


_acn_call = pl.pallas_call(
    _acn_body,
    grid=(_NBLK,),
    in_specs=[
        pl.BlockSpec((_BN, _D), lambda i: (i, 0)),
        pl.BlockSpec((1, 1, _BN), lambda i: (i, 0, 0)),
        pl.BlockSpec((_G, _D), lambda i: (0, 0)),
    ],
    out_specs=pl.BlockSpec((_BN, _D), lambda i: (i, 0)),
    out_shape=jax.ShapeDtypeStruct((_N, _D), _F32),
)


def _edge_body(ad, bs, w2, b2, wg, bg, wm_o, lgl_o):
    t = _lrelu(ad[...] + bs[...])
    m = _lrelu(_mm(t, w2[...]) + b2[...])
    g = jax.nn.sigmoid(_mm(m, wg[...]) + bg[...])
    lw = g * m
    wm_o[...] = lw * m
    lgl_o[...] = jnp.sqrt(jnp.sum(lw * lw, axis=1, keepdims=True))


_edge_call = pl.pallas_call(
    _edge_body,
    grid=(_EBLK,),
    in_specs=[
        pl.BlockSpec((_BE, _D), lambda i: (i, 0)),
        pl.BlockSpec((_BE, _D), lambda i: (i, 0)),
        pl.BlockSpec((_D, _D), lambda i: (0, 0)),
        pl.BlockSpec((1, _D), lambda i: (0, 0)),
        pl.BlockSpec((_D, _D), lambda i: (0, 0)),
        pl.BlockSpec((1, _D), lambda i: (0, 0)),
    ],
    out_specs=[
        pl.BlockSpec((_BE, _D), lambda i: (i, 0)),
        pl.BlockSpec((_BE, 1), lambda i: (i, 0)),
    ],
    out_shape=[
        jax.ShapeDtypeStruct((_E, _D), _F32),
        jax.ShapeDtypeStruct((_E, 1), _F32),
    ],
)


def _fin_body(gf, bng, bnb, w1, b1, w2, b2, out_o):
    g = gf[...]
    mu = jnp.mean(g, axis=0, keepdims=True)
    va = jnp.mean((g - mu) ** 2, axis=0, keepdims=True)
    xb = (g - mu) / jnp.sqrt(va + 1e-5) * bng[...] + bnb[...]
    h = _lrelu(_mm(xb, w1[...]) + b1[...])
    lo = _mm(h, w2[...]) + b2[...]
    mx = jnp.max(lo, axis=1, keepdims=True)
    lse = jnp.log(jnp.sum(jnp.exp(lo - mx), axis=1, keepdims=True)) + mx
    out_o[...] = lo - lse


_fin_call = pl.pallas_call(
    _fin_body,
    out_shape=jax.ShapeDtypeStruct((_G, _NCLS), _F32),
)


_sc_cache = {}


def _sc_kernels():
    if _sc_cache:
        return _sc_cache["gather"], _sc_cache["scatter"]
    mesh = plsc.VectorSubcoreMesh(core_axis_name="c", subcore_axis_name="s",
                                  num_cores=_NC, num_subcores=_NS)

    @functools.partial(
        pl.kernel,
        out_type=(
            jax.ShapeDtypeStruct((_E, _D), _F32),
            jax.ShapeDtypeStruct((_E, _D), _F32),
        ),
        mesh=mesh,
        scratch_types=[
            pltpu.VMEM((_K,), jnp.int32),
            pltpu.VMEM((_K,), jnp.int32),
            pltpu.VMEM((_K,), jnp.int32),
            pltpu.VMEM((_K,), jnp.int32),
            pltpu.VMEM((_K, _D), _F32),
            pltpu.VMEM((_K, _D), _F32),
            pltpu.VMEM((_K, _D), _F32),
            pltpu.VMEM((_K, _D), _F32),
        ] + [pltpu.SemaphoreType.DMA] * 10,
    )
    def gather_call(acn_hbm, bn_hbm, dst_hbm, src_hbm, ad_hbm, bs_hbm,
                    idd0, ids0, idd1, ids1, ra0, rb0, ra1, rb1,
                    si0, si1, sga0, sga1, sgb0, sgb1, swa0, swa1, swb0, swb1):
        c = lax.axis_index("c")
        s = lax.axis_index("s")
        base = (s * _NC + c) * _EPW

        def start_idx(ch, idd, ids, si):
            off = base + ch * _K
            pltpu.async_copy(dst_hbm.at[pl.ds(off, _K)], idd, si)
            pltpu.async_copy(src_hbm.at[pl.ds(off, _K)], ids, si)

        def wait_idx(idd, ids, si):
            pltpu.make_async_copy(dst_hbm.at[pl.ds(base, _K)], idd, si).wait()
            pltpu.make_async_copy(src_hbm.at[pl.ds(base, _K)], ids, si).wait()

        def start_gather(idd, ids, ra, rb, sa, sb):
            pltpu.async_copy(acn_hbm.at[idd], ra, sa)
            pltpu.async_copy(bn_hbm.at[ids], rb, sb)

        def wait_gather(idd, ids, ra, rb, sa, sb):
            pltpu.make_async_copy(acn_hbm.at[idd], ra, sa).wait()
            pltpu.make_async_copy(bn_hbm.at[ids], rb, sb).wait()

        def start_write(ch, ra, rb, sa, sb):
            off = base + ch * _K
            pltpu.async_copy(ra, ad_hbm.at[pl.ds(off, _K)], sa)
            pltpu.async_copy(rb, bs_hbm.at[pl.ds(off, _K)], sb)

        def wait_write(ra, rb, sa, sb):
            pltpu.make_async_copy(ra, ad_hbm.at[pl.ds(base, _K)], sa).wait()
            pltpu.make_async_copy(rb, bs_hbm.at[pl.ds(base, _K)], sb).wait()

        start_idx(0, idd0, ids0, si0)
        wait_idx(idd0, ids0, si0)
        start_gather(idd0, ids0, ra0, rb0, sga0, sgb0)
        start_idx(1, idd1, ids1, si1)

        def body(c2, carry):
            codd = 2 * c2 + 1
            wait_idx(idd1, ids1, si1)

            @pl.when(c2 > 0)
            def _():
                wait_write(ra1, rb1, swa1, swb1)

            start_gather(idd1, ids1, ra1, rb1, sga1, sgb1)
            wait_gather(idd0, ids0, ra0, rb0, sga0, sgb0)
            start_write(codd - 1, ra0, rb0, swa0, swb0)
            start_idx(codd + 1, idd0, ids0, si0)

            ceven = codd + 1
            wait_idx(idd0, ids0, si0)
            wait_write(ra0, rb0, swa0, swb0)
            start_gather(idd0, ids0, ra0, rb0, sga0, sgb0)
            wait_gather(idd1, ids1, ra1, rb1, sga1, sgb1)
            start_write(ceven - 1, ra1, rb1, swa1, swb1)

            @pl.when(c2 < _NCH2 - 1)
            def _():
                start_idx(ceven + 1, idd1, ids1, si1)

            return carry

        lax.fori_loop(0, _NCH2, body, 0)
        wait_gather(idd0, ids0, ra0, rb0, sga0, sgb0)
        start_write(_NCH - 1, ra0, rb0, swa0, swb0)
        wait_write(ra1, rb1, swa1, swb1)
        wait_write(ra0, rb0, swa0, swb0)

    @functools.partial(
        pl.kernel,
        out_type=jax.ShapeDtypeStruct((_NC, _NP, _D), _F32),
        mesh=mesh,
        scratch_types=[
            pltpu.VMEM((_K,), jnp.int32),
            pltpu.VMEM((_K,), jnp.int32),
            pltpu.VMEM((_K, _D), _F32),
            pltpu.VMEM((_K, _D), _F32),
            pltpu.VMEM((_RCH, _D), _F32),
            pltpu.VMEM_SHARED((_NP, _D), _F32),
        ] + [pltpu.SemaphoreType.DMA] * 4,
    )
    def scatter_call(wm_hbm, dst_hbm, out_hbm,
                     idx0, idx1, rows0, rows1, stage, acc,
                     sl0, sl1, ss0, ss1):
        c = lax.axis_index("c")
        s = lax.axis_index("s")
        base = (s * _NC + c) * _EPW

        zv = jnp.zeros((16,), _F32)

        def zrow(r, carry):
            for jj in range(_D // 16):
                stage[r, pl.ds(jj * 16, 16)] = zv
            return carry

        lax.fori_loop(0, _RCH, zrow, 0)
        for q in range(_RPT // _RCH):
            pltpu.sync_copy(stage, acc.at[pl.ds(s * _RPT + q * _RCH, _RCH)])
        _TAIL = _RPT - (_RPT // _RCH) * _RCH
        if _TAIL:
            pltpu.sync_copy(stage.at[pl.ds(0, _TAIL)],
                            acc.at[pl.ds(s * _RPT + _RPT - _TAIL, _TAIL)])
        plsc.subcore_barrier()

        def start_loads(ch, idx, rows, sl):
            off = base + ch * _K
            pltpu.async_copy(dst_hbm.at[pl.ds(off, _K)], idx, sl)
            pltpu.async_copy(wm_hbm.at[pl.ds(off, _K)], rows, sl)

        def wait_loads(idx, rows, sl):
            pltpu.make_async_copy(dst_hbm.at[pl.ds(base, _K)], idx, sl).wait()
            pltpu.make_async_copy(wm_hbm.at[pl.ds(base, _K)], rows, sl).wait()

        def wait_scat(idx, rows, ss):
            pltpu.make_async_copy(rows, acc.at[idx], ss).wait()

        start_loads(0, idx0, rows0, sl0)

        def body(c2, carry):
            wait_loads(idx0, rows0, sl0)
            pltpu.async_copy(rows0, acc.at[idx0], ss0, add=True)

            @pl.when(c2 > 0)
            def _():
                wait_scat(idx1, rows1, ss1)

            start_loads(2 * c2 + 1, idx1, rows1, sl1)
            wait_loads(idx1, rows1, sl1)
            pltpu.async_copy(rows1, acc.at[idx1], ss1, add=True)
            wait_scat(idx0, rows0, ss0)
            start_loads(2 * c2 + 2, idx0, rows0, sl0)
            return carry

        lax.fori_loop(0, _NCH2, body, 0)
        wait_loads(idx0, rows0, sl0)
        pltpu.async_copy(rows0, acc.at[idx0], ss0, add=True)
        wait_scat(idx1, rows1, ss1)
        wait_scat(idx0, rows0, ss0)
        plsc.subcore_barrier()

        for q in range(_RPT // _RCH):
            r0 = s * _RPT + q * _RCH
            pltpu.sync_copy(acc.at[pl.ds(r0, _RCH)], stage)
            pltpu.sync_copy(stage, out_hbm.at[c, pl.ds(r0, _RCH)])
        _TAIL2 = _RPT - (_RPT // _RCH) * _RCH
        if _TAIL2:
            r0 = s * _RPT + _RPT - _TAIL2
            pltpu.sync_copy(acc.at[pl.ds(r0, _TAIL2)], stage.at[pl.ds(0, _TAIL2)])
            pltpu.sync_copy(stage.at[pl.ds(0, _TAIL2)], out_hbm.at[c, pl.ds(r0, _TAIL2)])

    _sc_cache["gather"] = gather_call
    _sc_cache["scatter"] = scatter_call
    return gather_call, scatter_call


def kernel(x, edge_index, batch, params):
    p = params
    src = edge_index[0]
    dst = edge_index[1]
    batch3 = batch.reshape(_NBLK, 1, _BN)
    zp = jnp.zeros((_N, _D), _F32)
    rw1 = p['rd_W'][:_D]
    rw2 = p['rd_W'][_D:]
    rb = p['rd_b'].reshape(1, _D)
    bih = p['gru_bih'].reshape(1, 3 * _D)
    bhh = p['gru_bhh'].reshape(1, 3 * _D)

    x1 = _fc_call(x, p['fc_W'], p['fc_b'].reshape(1, _D))
    lgls = []
    ggls = []
    p0, p1 = x1, zp
    gf = jnp.zeros((_G, _D), _F32)
    for i in range(_NB):
        w1 = p[f'c{i}_m1_W']
        a_n, b_n, gfn, st = _ro_call(p0, p1, batch3, gf, rw1, rw2, rb,
                                     w1[:_D], w1[_D:2 * _D])
        gf2, c2, ggl = _gru_call(gfn, gf, st, p['gru_Wih'], bih,
                                 p['gru_Whh'], bhh, w1[2 * _D:],
                                 p[f'c{i}_m1_b'].reshape(1, _D))
        acn = _acn_call(a_n, batch3, c2)
        gather_fn, scatter_fn = _sc_kernels()
        ad, bs = gather_fn(acn, b_n, dst, src)
        wm, lgl = _edge_call(ad, bs, p[f'c{i}_m2_W'],
                             p[f'c{i}_m2_b'].reshape(1, _D),
                             p[f'c{i}_g_W'], p[f'c{i}_g_b'].reshape(1, _D))
        parts = scatter_fn(wm, dst)
        p0, p1 = parts[0, :_N], parts[1, :_N]
        gf = gf2
        lgls.append(lgl)
        ggls.append(ggl)

    w1 = p['c0_m1_W']
    _, _, gfn, st = _ro_call(p0, p1, batch3, gf, rw1, rw2, rb,
                             w1[:_D], w1[_D:2 * _D])
    gf_fin, _, ggl = _gru_call(gfn, gf, st, p['gru_Wih'], bih,
                               p['gru_Whh'], bhh, w1[2 * _D:],
                               p['c0_m1_b'].reshape(1, _D))
    ggls.append(ggl)

    out = _fin_call(gf_fin, p['bn_g'].reshape(1, _D), p['bn_b'].reshape(1, _D),
                    p['clf1_W'], p['clf1_b'].reshape(1, _HID),
                    p['clf2_W'], p['clf2_b'].reshape(1, _NCLS))
    lgl_cat = jnp.concatenate(lgls, axis=1)
    ggl_stack = jnp.concatenate([g.reshape(1) for g in ggls], axis=0)
    return out, lgl_cat, ggl_stack


# speedup vs baseline: 1.6184x; 1.6184x over previous
import functools

import jax
import jax.numpy as jnp
from jax import lax
from jax.experimental import pallas as pl
from jax.experimental.pallas import tpu as pltpu
from jax.experimental.pallas import tpu_sc as plsc

_N = 10000
_E = 320000
_D = 128
_G = 64
_NB = 3
_HID = 64
_NCLS = 10

_BN = 2000
_NBLK = _N // _BN

_NC = 2
_NS = 16
_NW = _NC * _NS
_K = 80
_EPW = _E // _NW
_NCH = _EPW // _K
_NCH2 = (_NCH - 1) // 2
_BE = 4000
_EBLK = _E // _BE
_NP = 10112
_RPT = _NP // _NS
_RCH = 128
_F32 = jnp.float32


def _mm(a, b):
    return jnp.dot(a, b, preferred_element_type=_F32)


def _lrelu(v):
    return jnp.where(v > 0, v, 0.01 * v)


def _fc_body(x_ref, w_ref, b_ref, o_ref):
    o_ref[...] = _mm(x_ref[...], w_ref[...]) + b_ref[...]


_fc_call = pl.pallas_call(
    _fc_body,
    grid=(_NBLK,),
    in_specs=[
        pl.BlockSpec((_BN, _D), lambda i: (i, 0)),
        pl.BlockSpec((_D, _D), lambda i: (0, 0)),
        pl.BlockSpec((1, _D), lambda i: (0, 0)),
    ],
    out_specs=pl.BlockSpec((_BN, _D), lambda i: (i, 0)),
    out_shape=jax.ShapeDtypeStruct((_N, _D), _F32),
)


def _ro_body(p0, p1, b3, gf, rw1, rw2, rb, w1a, w1b, a_o, b_o, gfn_o, st_o):
    i = pl.program_id(0)
    xb = p0[...] + p1[...]
    bt = b3[0, 0, :]
    oh = (bt[:, None] == lax.broadcasted_iota(jnp.int32, (_BN, _G), 1)).astype(_F32)
    oht = (lax.broadcasted_iota(jnp.int32, (_G, _BN), 0) == bt[None, :]).astype(_F32)
    gfb = _mm(oh, gf[...])
    gw = jax.nn.sigmoid(_mm(xb, rw1[...]) + _mm(gfb, rw2[...]) + rb[...])
    gfn_c = _mm(oht, gw * xb)
    nns = jnp.sqrt(jnp.sum(gw * gw, axis=1, keepdims=True))
    li = lax.broadcasted_iota(jnp.int32, (_BN, 8), 1)
    s8 = jnp.where(li == 0, nns, jnp.where(li == 1, 1.0, 0.0))
    st_c = _mm(oht, s8)
    a_o[...] = _mm(xb, w1a[...])
    b_o[...] = _mm(xb, w1b[...])

    @pl.when(i == 0)
    def _():
        gfn_o[...] = gfn_c
        st_o[...] = st_c

    @pl.when(i != 0)
    def _():
        gfn_o[...] += gfn_c
        st_o[...] += st_c


_ro_call = pl.pallas_call(
    _ro_body,
    grid=(_NBLK,),
    in_specs=[
        pl.BlockSpec((_BN, _D), lambda i: (i, 0)),
        pl.BlockSpec((_BN, _D), lambda i: (i, 0)),
        pl.BlockSpec((1, 1, _BN), lambda i: (i, 0, 0)),
        pl.BlockSpec((_G, _D), lambda i: (0, 0)),
        pl.BlockSpec((_D, _D), lambda i: (0, 0)),
        pl.BlockSpec((_D, _D), lambda i: (0, 0)),
        pl.BlockSpec((1, _D), lambda i: (0, 0)),
        pl.BlockSpec((_D, _D), lambda i: (0, 0)),
        pl.BlockSpec((_D, _D), lambda i: (0, 0)),
    ],
    out_specs=[
        pl.BlockSpec((_BN, _D), lambda i: (i, 0)),
        pl.BlockSpec((_BN, _D), lambda i: (i, 0)),
        pl.BlockSpec((_G, _D), lambda i: (0, 0)),
        pl.BlockSpec((_G, 8), lambda i: (0, 0)),
    ],
    out_shape=[
        jax.ShapeDtypeStruct((_N, _D), _F32),
        jax.ShapeDtypeStruct((_N, _D), _F32),
        jax.ShapeDtypeStruct((_G, _D), _F32),
        jax.ShapeDtypeStruct((_G, 8), _F32),
    ],
)


def _gru_body(gfn, gf, st, wih, bih, whh, bhh, w1c, b1, gf2_o, c2_o, ggl_o):
    gi = _mm(gfn[...], wih[...]) + bih[...]
    gh = _mm(gf[...], whh[...]) + bhh[...]
    r = jax.nn.sigmoid(gi[:, :_D] + gh[:, :_D])
    z = jax.nn.sigmoid(gi[:, _D:2 * _D] + gh[:, _D:2 * _D])
    n = jnp.tanh(gi[:, 2 * _D:] + r * gh[:, 2 * _D:])
    g2 = (1.0 - z) * n + z * gf[...]
    gf2_o[...] = g2
    c2_o[...] = _mm(g2, w1c[...]) + b1[...]
    s = st[...]
    val = jnp.mean(s[:, 0:1] / jnp.maximum(s[:, 1:2], 1.0))
    ggl_o[...] = val * jnp.ones((1, 1), _F32)


_gru_call = pl.pallas_call(
    _gru_body,
    out_shape=[
        jax.ShapeDtypeStruct((_G, _D), _F32),
        jax.ShapeDtypeStruct((_G, _D), _F32),
        jax.ShapeDtypeStruct((1, 1), _F32),
    ],
)


def _acn_body(a, b3, c2, o):
    bt = b3[0, 0, :]
    oh = (bt[:, None] == lax.broadcasted_iota(jnp.int32, (_BN, _G), 1)).astype(_F32)
    o[...] = a[...] + _mm(oh, c2[...])


_acn_call = pl.pallas_call(
    _acn_body,
    grid=(_NBLK,),
    in_specs=[
        pl.BlockSpec((_BN, _D), lambda i: (i, 0)),
        pl.BlockSpec((1, 1, _BN), lambda i: (i, 0, 0)),
        pl.BlockSpec((_G, _D), lambda i: (0, 0)),
    ],
    out_specs=pl.BlockSpec((_BN, _D), lambda i: (i, 0)),
    out_shape=jax.ShapeDtypeStruct((_N, _D), _F32),
)


def _edge_body(ad, bs, w2, b2, wg, bg, wm_o, lgl_o):
    t = _lrelu(ad[...] + bs[...])
    m = _lrelu(_mm(t, w2[...]) + b2[...])
    g = jax.nn.sigmoid(_mm(m, wg[...]) + bg[...])
    lw = g * m
    wm_o[...] = lw * m
    lgl_o[...] = jnp.sqrt(jnp.sum(lw * lw, axis=1, keepdims=True))


_edge_call = pl.pallas_call(
    _edge_body,
    grid=(_EBLK,),
    in_specs=[
        pl.BlockSpec((_BE, _D), lambda i: (i, 0)),
        pl.BlockSpec((_BE, _D), lambda i: (i, 0)),
        pl.BlockSpec((_D, _D), lambda i: (0, 0)),
        pl.BlockSpec((1, _D), lambda i: (0, 0)),
        pl.BlockSpec((_D, _D), lambda i: (0, 0)),
        pl.BlockSpec((1, _D), lambda i: (0, 0)),
    ],
    out_specs=[
        pl.BlockSpec((_BE, _D), lambda i: (i, 0)),
        pl.BlockSpec((_BE, 1), lambda i: (i, 0)),
    ],
    out_shape=[
        jax.ShapeDtypeStruct((_E, _D), _F32),
        jax.ShapeDtypeStruct((_E, 1), _F32),
    ],
)


def _fin_body(gf, bng, bnb, w1, b1, w2, b2, out_o):
    g = gf[...]
    mu = jnp.mean(g, axis=0, keepdims=True)
    va = jnp.mean((g - mu) ** 2, axis=0, keepdims=True)
    xb = (g - mu) / jnp.sqrt(va + 1e-5) * bng[...] + bnb[...]
    h = _lrelu(_mm(xb, w1[...]) + b1[...])
    lo = _mm(h, w2[...]) + b2[...]
    mx = jnp.max(lo, axis=1, keepdims=True)
    lse = jnp.log(jnp.sum(jnp.exp(lo - mx), axis=1, keepdims=True)) + mx
    out_o[...] = lo - lse


_fin_call = pl.pallas_call(
    _fin_body,
    out_shape=jax.ShapeDtypeStruct((_G, _NCLS), _F32),
)


_sc_cache = {}


def _sc_kernels():
    if _sc_cache:
        return _sc_cache["gather"], _sc_cache["scatter"]
    mesh = plsc.VectorSubcoreMesh(core_axis_name="c", subcore_axis_name="s",
                                  num_cores=_NC, num_subcores=_NS)

    @functools.partial(
        pl.kernel,
        out_type=(
            jax.ShapeDtypeStruct((_E, _D), _F32),
            jax.ShapeDtypeStruct((_E, _D), _F32),
        ),
        mesh=mesh,
        scratch_types=[
            pltpu.VMEM((_K,), jnp.int32),
            pltpu.VMEM((_K,), jnp.int32),
            pltpu.VMEM((_K,), jnp.int32),
            pltpu.VMEM((_K,), jnp.int32),
            pltpu.VMEM((_K, _D), _F32),
            pltpu.VMEM((_K, _D), _F32),
            pltpu.VMEM((_K, _D), _F32),
            pltpu.VMEM((_K, _D), _F32),
        ] + [pltpu.SemaphoreType.DMA] * 10,
    )
    def gather_call(acn_hbm, bn_hbm, dst_hbm, src_hbm, ad_hbm, bs_hbm,
                    idd0, ids0, idd1, ids1, ra0, rb0, ra1, rb1,
                    si0, si1, sga0, sga1, sgb0, sgb1, swa0, swa1, swb0, swb1):
        c = lax.axis_index("c")
        s = lax.axis_index("s")
        base = (s * _NC + c) * _EPW

        def start_idx(ch, idd, ids, si):
            off = base + ch * _K
            pltpu.async_copy(dst_hbm.at[pl.ds(off, _K)], idd, si)
            pltpu.async_copy(src_hbm.at[pl.ds(off, _K)], ids, si)

        def wait_idx(idd, ids, si):
            pltpu.make_async_copy(dst_hbm.at[pl.ds(base, _K)], idd, si).wait()
            pltpu.make_async_copy(src_hbm.at[pl.ds(base, _K)], ids, si).wait()

        def start_gather(idd, ids, ra, rb, sa, sb):
            pltpu.async_copy(acn_hbm.at[idd], ra, sa)
            pltpu.async_copy(bn_hbm.at[ids], rb, sb)

        def wait_gather(idd, ids, ra, rb, sa, sb):
            pltpu.make_async_copy(acn_hbm.at[idd], ra, sa).wait()
            pltpu.make_async_copy(bn_hbm.at[ids], rb, sb).wait()

        def start_write(ch, ra, rb, sa, sb):
            off = base + ch * _K
            pltpu.async_copy(ra, ad_hbm.at[pl.ds(off, _K)], sa)
            pltpu.async_copy(rb, bs_hbm.at[pl.ds(off, _K)], sb)

        def wait_write(ra, rb, sa, sb):
            pltpu.make_async_copy(ra, ad_hbm.at[pl.ds(base, _K)], sa).wait()
            pltpu.make_async_copy(rb, bs_hbm.at[pl.ds(base, _K)], sb).wait()

        start_idx(0, idd0, ids0, si0)
        wait_idx(idd0, ids0, si0)
        start_gather(idd0, ids0, ra0, rb0, sga0, sgb0)
        start_idx(1, idd1, ids1, si1)

        def body(c2, carry):
            codd = 2 * c2 + 1
            wait_idx(idd1, ids1, si1)

            @pl.when(c2 > 0)
            def _():
                wait_write(ra1, rb1, swa1, swb1)

            start_gather(idd1, ids1, ra1, rb1, sga1, sgb1)
            wait_gather(idd0, ids0, ra0, rb0, sga0, sgb0)
            start_write(codd - 1, ra0, rb0, swa0, swb0)
            start_idx(codd + 1, idd0, ids0, si0)

            ceven = codd + 1
            wait_idx(idd0, ids0, si0)
            wait_write(ra0, rb0, swa0, swb0)
            start_gather(idd0, ids0, ra0, rb0, sga0, sgb0)
            wait_gather(idd1, ids1, ra1, rb1, sga1, sgb1)
            start_write(ceven - 1, ra1, rb1, swa1, swb1)

            @pl.when(c2 < _NCH2 - 1)
            def _():
                start_idx(ceven + 1, idd1, ids1, si1)

            return carry

        lax.fori_loop(0, _NCH2, body, 0)
        wait_gather(idd0, ids0, ra0, rb0, sga0, sgb0)
        start_write(_NCH - 1, ra0, rb0, swa0, swb0)
        wait_write(ra1, rb1, swa1, swb1)
        wait_write(ra0, rb0, swa0, swb0)

    @functools.partial(
        pl.kernel,
        out_type=jax.ShapeDtypeStruct((_NC, _NP, _D), _F32),
        mesh=mesh,
        scratch_types=[
            pltpu.VMEM((_K,), jnp.int32),
            pltpu.VMEM((_K,), jnp.int32),
            pltpu.VMEM((_K, _D), _F32),
            pltpu.VMEM((_K, _D), _F32),
            pltpu.VMEM((_RCH, _D), _F32),
            pltpu.VMEM_SHARED((_NP, _D), _F32),
        ] + [pltpu.SemaphoreType.DMA] * 4,
    )
    def scatter_call(wm_hbm, dst_hbm, out_hbm,
                     idx0, idx1, rows0, rows1, stage, acc,
                     sl0, sl1, ss0, ss1):
        c = lax.axis_index("c")
        s = lax.axis_index("s")
        base = (s * _NC + c) * _EPW

        zv = jnp.zeros((16,), _F32)

        def zrow(r, carry):
            for jj in range(_D // 16):
                stage[r, pl.ds(jj * 16, 16)] = zv
            return carry

        lax.fori_loop(0, _RCH, zrow, 0)
        for q in range(_RPT // _RCH):
            pltpu.sync_copy(stage, acc.at[pl.ds(s * _RPT + q * _RCH, _RCH)])
        _TAIL = _RPT - (_RPT // _RCH) * _RCH
        if _TAIL:
            pltpu.sync_copy(stage.at[pl.ds(0, _TAIL)],
                            acc.at[pl.ds(s * _RPT + _RPT - _TAIL, _TAIL)])
        plsc.subcore_barrier()

        def start_loads(ch, idx, rows, sl):
            off = base + ch * _K
            pltpu.async_copy(dst_hbm.at[pl.ds(off, _K)], idx, sl)
            pltpu.async_copy(wm_hbm.at[pl.ds(off, _K)], rows, sl)

        def wait_loads(idx, rows, sl):
            pltpu.make_async_copy(dst_hbm.at[pl.ds(base, _K)], idx, sl).wait()
            pltpu.make_async_copy(wm_hbm.at[pl.ds(base, _K)], rows, sl).wait()

        def wait_scat(idx, rows, ss):
            pltpu.make_async_copy(rows, acc.at[idx], ss).wait()

        start_loads(0, idx0, rows0, sl0)

        def body(c2, carry):
            wait_loads(idx0, rows0, sl0)
            pltpu.async_copy(rows0, acc.at[idx0], ss0, add=True)

            @pl.when(c2 > 0)
            def _():
                wait_scat(idx1, rows1, ss1)

            start_loads(2 * c2 + 1, idx1, rows1, sl1)
            wait_loads(idx1, rows1, sl1)
            pltpu.async_copy(rows1, acc.at[idx1], ss1, add=True)
            wait_scat(idx0, rows0, ss0)
            start_loads(2 * c2 + 2, idx0, rows0, sl0)
            return carry

        lax.fori_loop(0, _NCH2, body, 0)
        wait_loads(idx0, rows0, sl0)
        pltpu.async_copy(rows0, acc.at[idx0], ss0, add=True)
        wait_scat(idx1, rows1, ss1)
        wait_scat(idx0, rows0, ss0)
        plsc.subcore_barrier()

        for q in range(_RPT // _RCH):
            r0 = s * _RPT + q * _RCH
            pltpu.sync_copy(acc.at[pl.ds(r0, _RCH)], stage)
            pltpu.sync_copy(stage, out_hbm.at[c, pl.ds(r0, _RCH)])
        _TAIL2 = _RPT - (_RPT // _RCH) * _RCH
        if _TAIL2:
            r0 = s * _RPT + _RPT - _TAIL2
            pltpu.sync_copy(acc.at[pl.ds(r0, _TAIL2)], stage.at[pl.ds(0, _TAIL2)])
            pltpu.sync_copy(stage.at[pl.ds(0, _TAIL2)], out_hbm.at[c, pl.ds(r0, _TAIL2)])

    _sc_cache["gather"] = gather_call
    _sc_cache["scatter"] = scatter_call
    return gather_call, scatter_call


def kernel(x, edge_index, batch, params):
    p = params
    src = edge_index[0]
    dst = edge_index[1]
    batch3 = batch.reshape(_NBLK, 1, _BN)
    zp = jnp.zeros((_N, _D), _F32)
    rw1 = p['rd_W'][:_D]
    rw2 = p['rd_W'][_D:]
    rb = p['rd_b'].reshape(1, _D)
    bih = p['gru_bih'].reshape(1, 3 * _D)
    bhh = p['gru_bhh'].reshape(1, 3 * _D)

    x1 = _fc_call(x, p['fc_W'], p['fc_b'].reshape(1, _D))
    lgls = []
    ggls = []
    p0, p1 = x1, zp
    gf = jnp.zeros((_G, _D), _F32)
    for i in range(_NB):
        w1 = p[f'c{i}_m1_W']
        a_n, b_n, gfn, st = _ro_call(p0, p1, batch3, gf, rw1, rw2, rb,
                                     w1[:_D], w1[_D:2 * _D])
        gf2, c2, ggl = _gru_call(gfn, gf, st, p['gru_Wih'], bih,
                                 p['gru_Whh'], bhh, w1[2 * _D:],
                                 p[f'c{i}_m1_b'].reshape(1, _D))
        acn = _acn_call(a_n, batch3, c2)
        gather_fn, scatter_fn = _sc_kernels()
        ad, bs = gather_fn(acn, b_n, dst, src)
        wm, lgl = _edge_call(ad, bs, p[f'c{i}_m2_W'],
                             p[f'c{i}_m2_b'].reshape(1, _D),
                             p[f'c{i}_g_W'], p[f'c{i}_g_b'].reshape(1, _D))
        parts = scatter_fn(wm, dst)
        p0, p1 = parts[0, :_N], parts[1, :_N]
        gf = gf2
        lgls.append(lgl)
        ggls.append(ggl)

    w1 = p['c0_m1_W']
    _, _, gfn, st = _ro_call(p0, p1, batch3, gf, rw1, rw2, rb,
                             w1[:_D], w1[_D:2 * _D])
    gf_fin, _, ggl = _gru_call(gfn, gf, st, p['gru_Wih'], bih,
                               p['gru_Whh'], bhh, w1[2 * _D:],
                               p['c0_m1_b'].reshape(1, _D))
    ggls.append(ggl)

    out = _fin_call(gf_fin, p['bn_g'].reshape(1, _D), p['bn_b'].reshape(1, _D),
                    p['clf1_W'], p['clf1_b'].reshape(1, _HID),
                    p['clf2_W'], p['clf2_b'].reshape(1, _NCLS))
    lgl_cat = jnp.concatenate(lgls, axis=1)
    ggl_stack = jnp.concatenate([g.reshape(1) for g in ggls], axis=0)
    return out, lgl_cat, ggl_stack


# speedup vs baseline: 1.6509x; 1.0201x over previous
import functools

import jax
import jax.numpy as jnp
from jax import lax
from jax.experimental import pallas as pl
from jax.experimental.pallas import tpu as pltpu
from jax.experimental.pallas import tpu_sc as plsc

_N = 10000
_E = 320000
_D = 128
_G = 64
_NB = 3
_HID = 64
_NCLS = 10

_BN = 2000
_NBLK = _N // _BN

_NC = 2
_NS = 16
_NW = _NC * _NS
_K = 80
_EPW = _E // _NW
_NCH = _EPW // _K
_NCH2 = (_NCH - 1) // 2
_BE = 8000
_EBLK = _E // _BE
_NP = 10112
_RPT = _NP // _NS
_RCH = 128
_F32 = jnp.float32


def _mm(a, b):
    return jnp.dot(a, b, preferred_element_type=_F32)


def _lrelu(v):
    return jnp.where(v > 0, v, 0.01 * v)


def _fc_body(x_ref, w_ref, b_ref, o_ref):
    o_ref[...] = _mm(x_ref[...], w_ref[...]) + b_ref[...]


_fc_call = pl.pallas_call(
    _fc_body,
    grid=(_NBLK,),
    in_specs=[
        pl.BlockSpec((_BN, _D), lambda i: (i, 0)),
        pl.BlockSpec((_D, _D), lambda i: (0, 0)),
        pl.BlockSpec((1, _D), lambda i: (0, 0)),
    ],
    out_specs=pl.BlockSpec((_BN, _D), lambda i: (i, 0)),
    out_shape=jax.ShapeDtypeStruct((_N, _D), _F32),
)


def _ro_body(p0, p1, b3, gf, rw1, rw2, rb, w1a, w1b, a_o, b_o, gfn_o, st_o):
    i = pl.program_id(0)
    xb = p0[...] + p1[...]
    bt = b3[0, 0, :]
    oh = (bt[:, None] == lax.broadcasted_iota(jnp.int32, (_BN, _G), 1)).astype(_F32)
    oht = (lax.broadcasted_iota(jnp.int32, (_G, _BN), 0) == bt[None, :]).astype(_F32)
    gfb = _mm(oh, gf[...])
    gw = jax.nn.sigmoid(_mm(xb, rw1[...]) + _mm(gfb, rw2[...]) + rb[...])
    gfn_c = _mm(oht, gw * xb)
    nns = jnp.sqrt(jnp.sum(gw * gw, axis=1, keepdims=True))
    li = lax.broadcasted_iota(jnp.int32, (_BN, 8), 1)
    s8 = jnp.where(li == 0, nns, jnp.where(li == 1, 1.0, 0.0))
    st_c = _mm(oht, s8)
    a_o[...] = _mm(xb, w1a[...])
    b_o[...] = _mm(xb, w1b[...])

    @pl.when(i == 0)
    def _():
        gfn_o[...] = gfn_c
        st_o[...] = st_c

    @pl.when(i != 0)
    def _():
        gfn_o[...] += gfn_c
        st_o[...] += st_c


_ro_call = pl.pallas_call(
    _ro_body,
    grid=(_NBLK,),
    in_specs=[
        pl.BlockSpec((_BN, _D), lambda i: (i, 0)),
        pl.BlockSpec((_BN, _D), lambda i: (i, 0)),
        pl.BlockSpec((1, 1, _BN), lambda i: (i, 0, 0)),
        pl.BlockSpec((_G, _D), lambda i: (0, 0)),
        pl.BlockSpec((_D, _D), lambda i: (0, 0)),
        pl.BlockSpec((_D, _D), lambda i: (0, 0)),
        pl.BlockSpec((1, _D), lambda i: (0, 0)),
        pl.BlockSpec((_D, _D), lambda i: (0, 0)),
        pl.BlockSpec((_D, _D), lambda i: (0, 0)),
    ],
    out_specs=[
        pl.BlockSpec((_BN, _D), lambda i: (i, 0)),
        pl.BlockSpec((_BN, _D), lambda i: (i, 0)),
        pl.BlockSpec((_G, _D), lambda i: (0, 0)),
        pl.BlockSpec((_G, 8), lambda i: (0, 0)),
    ],
    out_shape=[
        jax.ShapeDtypeStruct((_N, _D), _F32),
        jax.ShapeDtypeStruct((_N, _D), _F32),
        jax.ShapeDtypeStruct((_G, _D), _F32),
        jax.ShapeDtypeStruct((_G, 8), _F32),
    ],
)


def _gru_body(gfn, gf, st, wih, bih, whh, bhh, w1c, b1, gf2_o, c2_o, ggl_o):
    gi = _mm(gfn[...], wih[...]) + bih[...]
    gh = _mm(gf[...], whh[...]) + bhh[...]
    r = jax.nn.sigmoid(gi[:, :_D] + gh[:, :_D])
    z = jax.nn.sigmoid(gi[:, _D:2 * _D] + gh[:, _D:2 * _D])
    n = jnp.tanh(gi[:, 2 * _D:] + r * gh[:, 2 * _D:])
    g2 = (1.0 - z) * n + z * gf[...]
    gf2_o[...] = g2
    c2_o[...] = _mm(g2, w1c[...]) + b1[...]
    s = st[...]
    val = jnp.mean(s[:, 0:1] / jnp.maximum(s[:, 1:2], 1.0))
    ggl_o[...] = val * jnp.ones((1, 1), _F32)


_gru_call = pl.pallas_call(
    _gru_body,
    out_shape=[
        jax.ShapeDtypeStruct((_G, _D), _F32),
        jax.ShapeDtypeStruct((_G, _D), _F32),
        jax.ShapeDtypeStruct((1, 1), _F32),
    ],
)


def _acn_body(a, b3, c2, o):
    bt = b3[0, 0, :]
    oh = (bt[:, None] == lax.broadcasted_iota(jnp.int32, (_BN, _G), 1)).astype(_F32)
    o[...] = a[...] + _mm(oh, c2[...])


_acn_call = pl.pallas_call(
    _acn_body,
    grid=(_NBLK,),
    in_specs=[
        pl.BlockSpec((_BN, _D), lambda i: (i, 0)),
        pl.BlockSpec((1, 1, _BN), lambda i: (i, 0, 0)),
        pl.BlockSpec((_G, _D), lambda i: (0, 0)),
    ],
    out_specs=pl.BlockSpec((_BN, _D), lambda i: (i, 0)),
    out_shape=jax.ShapeDtypeStruct((_N, _D), _F32),
)


def _edge_body(ad, bs, w2, b2, wg, bg, wm_o, lgl_o):
    t = _lrelu(ad[...] + bs[...])
    m = _lrelu(_mm(t, w2[...]) + b2[...])
    g = jax.nn.sigmoid(_mm(m, wg[...]) + bg[...])
    lw = g * m
    wm_o[...] = lw * m
    lgl_o[...] = jnp.sqrt(jnp.sum(lw * lw, axis=1, keepdims=True))


_edge_call = pl.pallas_call(
    _edge_body,
    grid=(_EBLK,),
    in_specs=[
        pl.BlockSpec((_BE, _D), lambda i: (i, 0)),
        pl.BlockSpec((_BE, _D), lambda i: (i, 0)),
        pl.BlockSpec((_D, _D), lambda i: (0, 0)),
        pl.BlockSpec((1, _D), lambda i: (0, 0)),
        pl.BlockSpec((_D, _D), lambda i: (0, 0)),
        pl.BlockSpec((1, _D), lambda i: (0, 0)),
    ],
    out_specs=[
        pl.BlockSpec((_BE, _D), lambda i: (i, 0)),
        pl.BlockSpec((_BE, 1), lambda i: (i, 0)),
    ],
    out_shape=[
        jax.ShapeDtypeStruct((_E, _D), _F32),
        jax.ShapeDtypeStruct((_E, 1), _F32),
    ],
)


def _fin_body(gf, bng, bnb, w1, b1, w2, b2, out_o):
    g = gf[...]
    mu = jnp.mean(g, axis=0, keepdims=True)
    va = jnp.mean((g - mu) ** 2, axis=0, keepdims=True)
    xb = (g - mu) / jnp.sqrt(va + 1e-5) * bng[...] + bnb[...]
    h = _lrelu(_mm(xb, w1[...]) + b1[...])
    lo = _mm(h, w2[...]) + b2[...]
    mx = jnp.max(lo, axis=1, keepdims=True)
    lse = jnp.log(jnp.sum(jnp.exp(lo - mx), axis=1, keepdims=True)) + mx
    out_o[...] = lo - lse


_fin_call = pl.pallas_call(
    _fin_body,
    out_shape=jax.ShapeDtypeStruct((_G, _NCLS), _F32),
)


_sc_cache = {}


def _sc_kernels():
    if _sc_cache:
        return _sc_cache["gather"], _sc_cache["scatter"]
    mesh = plsc.VectorSubcoreMesh(core_axis_name="c", subcore_axis_name="s",
                                  num_cores=_NC, num_subcores=_NS)

    @functools.partial(
        pl.kernel,
        out_type=(
            jax.ShapeDtypeStruct((_E, _D), _F32),
            jax.ShapeDtypeStruct((_E, _D), _F32),
        ),
        mesh=mesh,
        scratch_types=[
            pltpu.VMEM((_K,), jnp.int32),
            pltpu.VMEM((_K,), jnp.int32),
            pltpu.VMEM((_K,), jnp.int32),
            pltpu.VMEM((_K,), jnp.int32),
            pltpu.VMEM((_K, _D), _F32),
            pltpu.VMEM((_K, _D), _F32),
            pltpu.VMEM((_K, _D), _F32),
            pltpu.VMEM((_K, _D), _F32),
        ] + [pltpu.SemaphoreType.DMA] * 10,
    )
    def gather_call(acn_hbm, bn_hbm, dst_hbm, src_hbm, ad_hbm, bs_hbm,
                    idd0, ids0, idd1, ids1, ra0, rb0, ra1, rb1,
                    si0, si1, sga0, sga1, sgb0, sgb1, swa0, swa1, swb0, swb1):
        c = lax.axis_index("c")
        s = lax.axis_index("s")
        base = (s * _NC + c) * _EPW

        def start_idx(ch, idd, ids, si):
            off = base + ch * _K
            pltpu.async_copy(dst_hbm.at[pl.ds(off, _K)], idd, si)
            pltpu.async_copy(src_hbm.at[pl.ds(off, _K)], ids, si)

        def wait_idx(idd, ids, si):
            pltpu.make_async_copy(dst_hbm.at[pl.ds(base, _K)], idd, si).wait()
            pltpu.make_async_copy(src_hbm.at[pl.ds(base, _K)], ids, si).wait()

        def start_gather(idd, ids, ra, rb, sa, sb):
            pltpu.async_copy(acn_hbm.at[idd], ra, sa)
            pltpu.async_copy(bn_hbm.at[ids], rb, sb)

        def wait_gather(idd, ids, ra, rb, sa, sb):
            pltpu.make_async_copy(acn_hbm.at[idd], ra, sa).wait()
            pltpu.make_async_copy(bn_hbm.at[ids], rb, sb).wait()

        def start_write(ch, ra, rb, sa, sb):
            off = base + ch * _K
            pltpu.async_copy(ra, ad_hbm.at[pl.ds(off, _K)], sa)
            pltpu.async_copy(rb, bs_hbm.at[pl.ds(off, _K)], sb)

        def wait_write(ra, rb, sa, sb):
            pltpu.make_async_copy(ra, ad_hbm.at[pl.ds(base, _K)], sa).wait()
            pltpu.make_async_copy(rb, bs_hbm.at[pl.ds(base, _K)], sb).wait()

        start_idx(0, idd0, ids0, si0)
        wait_idx(idd0, ids0, si0)
        start_gather(idd0, ids0, ra0, rb0, sga0, sgb0)
        start_idx(1, idd1, ids1, si1)

        def body(c2, carry):
            codd = 2 * c2 + 1
            wait_idx(idd1, ids1, si1)

            @pl.when(c2 > 0)
            def _():
                wait_write(ra1, rb1, swa1, swb1)

            start_gather(idd1, ids1, ra1, rb1, sga1, sgb1)
            wait_gather(idd0, ids0, ra0, rb0, sga0, sgb0)
            start_write(codd - 1, ra0, rb0, swa0, swb0)
            start_idx(codd + 1, idd0, ids0, si0)

            ceven = codd + 1
            wait_idx(idd0, ids0, si0)
            wait_write(ra0, rb0, swa0, swb0)
            start_gather(idd0, ids0, ra0, rb0, sga0, sgb0)
            wait_gather(idd1, ids1, ra1, rb1, sga1, sgb1)
            start_write(ceven - 1, ra1, rb1, swa1, swb1)

            @pl.when(c2 < _NCH2 - 1)
            def _():
                start_idx(ceven + 1, idd1, ids1, si1)

            return carry

        lax.fori_loop(0, _NCH2, body, 0)
        wait_gather(idd0, ids0, ra0, rb0, sga0, sgb0)
        start_write(_NCH - 1, ra0, rb0, swa0, swb0)
        wait_write(ra1, rb1, swa1, swb1)
        wait_write(ra0, rb0, swa0, swb0)

    @functools.partial(
        pl.kernel,
        out_type=jax.ShapeDtypeStruct((_NC, _NP, _D), _F32),
        mesh=mesh,
        scratch_types=[
            pltpu.VMEM((_K,), jnp.int32),
            pltpu.VMEM((_K,), jnp.int32),
            pltpu.VMEM((_K, _D), _F32),
            pltpu.VMEM((_K, _D), _F32),
            pltpu.VMEM((_RCH, _D), _F32),
            pltpu.VMEM_SHARED((_NP, _D), _F32),
        ] + [pltpu.SemaphoreType.DMA] * 4,
    )
    def scatter_call(wm_hbm, dst_hbm, out_hbm,
                     idx0, idx1, rows0, rows1, stage, acc,
                     sl0, sl1, ss0, ss1):
        c = lax.axis_index("c")
        s = lax.axis_index("s")
        base = (s * _NC + c) * _EPW

        zv = jnp.zeros((16,), _F32)

        def zrow(r, carry):
            for jj in range(_D // 16):
                stage[r, pl.ds(jj * 16, 16)] = zv
            return carry

        lax.fori_loop(0, _RCH, zrow, 0)
        for q in range(_RPT // _RCH):
            pltpu.sync_copy(stage, acc.at[pl.ds(s * _RPT + q * _RCH, _RCH)])
        _TAIL = _RPT - (_RPT // _RCH) * _RCH
        if _TAIL:
            pltpu.sync_copy(stage.at[pl.ds(0, _TAIL)],
                            acc.at[pl.ds(s * _RPT + _RPT - _TAIL, _TAIL)])
        plsc.subcore_barrier()

        def start_loads(ch, idx, rows, sl):
            off = base + ch * _K
            pltpu.async_copy(dst_hbm.at[pl.ds(off, _K)], idx, sl)
            pltpu.async_copy(wm_hbm.at[pl.ds(off, _K)], rows, sl)

        def wait_loads(idx, rows, sl):
            pltpu.make_async_copy(dst_hbm.at[pl.ds(base, _K)], idx, sl).wait()
            pltpu.make_async_copy(wm_hbm.at[pl.ds(base, _K)], rows, sl).wait()

        def wait_scat(idx, rows, ss):
            pltpu.make_async_copy(rows, acc.at[idx], ss).wait()

        start_loads(0, idx0, rows0, sl0)

        def body(c2, carry):
            wait_loads(idx0, rows0, sl0)
            pltpu.async_copy(rows0, acc.at[idx0], ss0, add=True)

            @pl.when(c2 > 0)
            def _():
                wait_scat(idx1, rows1, ss1)

            start_loads(2 * c2 + 1, idx1, rows1, sl1)
            wait_loads(idx1, rows1, sl1)
            pltpu.async_copy(rows1, acc.at[idx1], ss1, add=True)
            wait_scat(idx0, rows0, ss0)
            start_loads(2 * c2 + 2, idx0, rows0, sl0)
            return carry

        lax.fori_loop(0, _NCH2, body, 0)
        wait_loads(idx0, rows0, sl0)
        pltpu.async_copy(rows0, acc.at[idx0], ss0, add=True)
        wait_scat(idx1, rows1, ss1)
        wait_scat(idx0, rows0, ss0)
        plsc.subcore_barrier()

        for q in range(_RPT // _RCH):
            r0 = s * _RPT + q * _RCH
            pltpu.sync_copy(acc.at[pl.ds(r0, _RCH)], stage)
            pltpu.sync_copy(stage, out_hbm.at[c, pl.ds(r0, _RCH)])
        _TAIL2 = _RPT - (_RPT // _RCH) * _RCH
        if _TAIL2:
            r0 = s * _RPT + _RPT - _TAIL2
            pltpu.sync_copy(acc.at[pl.ds(r0, _TAIL2)], stage.at[pl.ds(0, _TAIL2)])
            pltpu.sync_copy(stage.at[pl.ds(0, _TAIL2)], out_hbm.at[c, pl.ds(r0, _TAIL2)])

    _sc_cache["gather"] = gather_call
    _sc_cache["scatter"] = scatter_call
    return gather_call, scatter_call


def kernel(x, edge_index, batch, params):
    p = params
    src = edge_index[0]
    dst = edge_index[1]
    batch3 = batch.reshape(_NBLK, 1, _BN)
    zp = jnp.zeros((_N, _D), _F32)
    rw1 = p['rd_W'][:_D]
    rw2 = p['rd_W'][_D:]
    rb = p['rd_b'].reshape(1, _D)
    bih = p['gru_bih'].reshape(1, 3 * _D)
    bhh = p['gru_bhh'].reshape(1, 3 * _D)

    x1 = _fc_call(x, p['fc_W'], p['fc_b'].reshape(1, _D))
    lgls = []
    ggls = []
    p0, p1 = x1, zp
    gf = jnp.zeros((_G, _D), _F32)
    for i in range(_NB):
        w1 = p[f'c{i}_m1_W']
        a_n, b_n, gfn, st = _ro_call(p0, p1, batch3, gf, rw1, rw2, rb,
                                     w1[:_D], w1[_D:2 * _D])
        gf2, c2, ggl = _gru_call(gfn, gf, st, p['gru_Wih'], bih,
                                 p['gru_Whh'], bhh, w1[2 * _D:],
                                 p[f'c{i}_m1_b'].reshape(1, _D))
        acn = _acn_call(a_n, batch3, c2)
        gather_fn, scatter_fn = _sc_kernels()
        ad, bs = gather_fn(acn, b_n, dst, src)
        wm, lgl = _edge_call(ad, bs, p[f'c{i}_m2_W'],
                             p[f'c{i}_m2_b'].reshape(1, _D),
                             p[f'c{i}_g_W'], p[f'c{i}_g_b'].reshape(1, _D))
        parts = scatter_fn(wm, dst)
        p0, p1 = parts[0, :_N], parts[1, :_N]
        gf = gf2
        lgls.append(lgl)
        ggls.append(ggl)

    w1 = p['c0_m1_W']
    _, _, gfn, st = _ro_call(p0, p1, batch3, gf, rw1, rw2, rb,
                             w1[:_D], w1[_D:2 * _D])
    gf_fin, _, ggl = _gru_call(gfn, gf, st, p['gru_Wih'], bih,
                               p['gru_Whh'], bhh, w1[2 * _D:],
                               p['c0_m1_b'].reshape(1, _D))
    ggls.append(ggl)

    out = _fin_call(gf_fin, p['bn_g'].reshape(1, _D), p['bn_b'].reshape(1, _D),
                    p['clf1_W'], p['clf1_b'].reshape(1, _HID),
                    p['clf2_W'], p['clf2_b'].reshape(1, _NCLS))
    lgl_cat = jnp.concatenate(lgls, axis=1)
    ggl_stack = jnp.concatenate([g.reshape(1) for g in ggls], axis=0)
    return out, lgl_cat, ggl_stack
